# baseline (speedup 1.0000x reference)
import jax
import jax.numpy as jnp
from jax import lax
from jax.experimental import pallas as pl
from jax.experimental.pallas import tpu as pltpu
from jax.experimental.pallas import tpu_sc as plsc

VOCAB = 1000000
EMBED = 64
B = 16384
K = 20
KPAD = 32

NC = 2
NS = 16
NW = NC * NS
NB = B // NW
C = 32
NCHUNK = NB // C
ROWS = C * K
GID = 128
NG = ROWS // GID


def _sc_score_kernel(center_ref, ctx_ref, in_emb_ref, out_emb_ref,
                     score_ref, cidx_v, ctxidx_v, crows_v, rows_v,
                     score_v, sem):
    wid = lax.axis_index("s") * NC + lax.axis_index("c")
    lane = lax.iota(jnp.int32, 16)

    pltpu.sync_copy(center_ref.at[pl.ds(wid * 4, 4)], cidx_v)
    pltpu.sync_copy(ctx_ref.at[pl.ds(wid * (NB * K // 128), NB * K // 128)],
                    ctxidx_v)

    cdescs = [
        pltpu.async_copy(in_emb_ref.at[cidx_v.at[t]],
                         crows_v.at[pl.ds(t * GID, GID)], sem)
        for t in range(4)
    ]
    for d in cdescs:
        d.wait()

    def chunk_body(j, carry):
        descs = [
            pltpu.async_copy(out_emb_ref.at[ctxidx_v.at[j * NG + t]],
                             rows_v.at[pl.ds(t * GID, GID)], sem)
            for t in range(NG)
        ]
        for d in descs:
            d.wait()

        def center_body(c2, carry2):
            cg = j * C + c2
            cv = [crows_v[cg, pl.ds(t * 16, 16)] for t in range(4)]
            s_lo = jnp.zeros((16,), jnp.float32)
            s_hi = jnp.full((16,), -1e30, jnp.float32)
            for k in range(K):
                r0 = rows_v[c2 * K + k, pl.ds(0, 16)]
                r1 = rows_v[c2 * K + k, pl.ds(16, 16)]
                r2 = rows_v[c2 * K + k, pl.ds(32, 16)]
                r3 = rows_v[c2 * K + k, pl.ds(48, 16)]
                p = cv[0] * r0 + cv[1] * r1 + cv[2] * r2 + cv[3] * r3
                s = jnp.sum(p)
                if k < 16:
                    s_lo = jnp.where(lane == k, s, s_lo)
                else:
                    s_hi = jnp.where(lane == (k - 16), s, s_hi)
            score_v[c2, pl.ds(0, 16)] = s_lo
            score_v[c2, pl.ds(16, 16)] = s_hi
            return carry2

        lax.fori_loop(0, C, center_body, 0)
        pltpu.sync_copy(score_v,
                        score_ref.at[pl.ds(wid * NB + j * C, C)])
        return carry

    lax.fori_loop(0, NCHUNK, chunk_body, 0)


def _sc_score(center2d, ctx2d, in_embed, out_embed):
    mesh = plsc.VectorSubcoreMesh(core_axis_name="c", subcore_axis_name="s")
    f = pl.kernel(
        _sc_score_kernel,
        out_type=jax.ShapeDtypeStruct((B, KPAD), jnp.float32),
        mesh=mesh,
        scratch_types=[
            pltpu.VMEM((4, 128), jnp.int32),
            pltpu.VMEM((NB * K // 128, 128), jnp.int32),
            pltpu.VMEM((NB, EMBED), jnp.float32),
            pltpu.VMEM((ROWS, EMBED), jnp.float32),
            pltpu.VMEM((C, KPAD), jnp.float32),
            pltpu.SemaphoreType.DMA,
        ],
        compiler_params=pltpu.CompilerParams(
            needs_layout_passes=False, use_tc_tiling_on_sc=False
        ),
    )
    return f(center2d, ctx2d, in_embed, out_embed)


def _tc_loss_kernel(score_ref, out_ref):
    s = score_ref[...]
    m = jnp.max(s, axis=1, keepdims=True)
    e = jnp.exp(s - m)
    lse = m + jnp.log(jnp.sum(e, axis=1, keepdims=True))
    col = lax.broadcasted_iota(jnp.int32, (B, KPAD), 1)
    ssum = jnp.sum(jnp.where(col < K, s, 0.0))
    out_ref[...] = jnp.reshape(jnp.sum(lse) / B - ssum / (B * K), (1, 1))


def _tc_loss(score):
    return pl.pallas_call(
        _tc_loss_kernel,
        out_shape=jax.ShapeDtypeStruct((1, 1), jnp.float32),
    )(score)


def kernel(center_ids, context_ids, in_embed, out_embed):
    center2d = center_ids.astype(jnp.int32).reshape(B // 128, 128)
    ctx2d = context_ids.astype(jnp.int32).reshape(B * K // 128, 128)
    score = _sc_score(center2d, ctx2d, in_embed, out_embed)
    loss = _tc_loss(score)
    return loss[0, 0]
